# Initial kernel scaffold; baseline (speedup 1.0000x reference)
import jax
import jax.numpy as jnp
from jax.experimental import pallas as pl


def kernel(x, neigh, W, b):
    raise NotImplementedError("write your pallas kernel here")


# speedup vs baseline: 1.4308x; 1.4308x over previous
import functools

import jax
import jax.numpy as jnp
from jax import lax
from jax.experimental import pallas as pl
from jax.experimental.pallas import tpu as pltpu
from jax.experimental.pallas import tpu_sc as plsc

N = 10000
DEG = 32
F = 128
L = 16

NC, NS = 2, 16
NW = NC * NS
NP = 10240
PW = NP // NW
CN = 16
NCH = PW // CN

BM = 1000


def _mm_body(x_ref, w_ref, b_ref, o_ref):
    o_ref[...] = (
        jnp.dot(x_ref[...], w_ref[...], preferred_element_type=jnp.float32)
        + b_ref[...]
    )


def _matmul(x, W, b):
    return pl.pallas_call(
        _mm_body,
        grid=(N // BM,),
        in_specs=[
            pl.BlockSpec((BM, F), lambda i: (i, 0)),
            pl.BlockSpec((F, F), lambda i: (0, 0)),
            pl.BlockSpec((1, F), lambda i: (0, 0)),
        ],
        out_specs=pl.BlockSpec((BM, F), lambda i: (i, 0)),
        out_shape=jax.ShapeDtypeStruct((N, F), jnp.float32),
    )(x, W, b.reshape(1, F))


_sc_mesh = plsc.VectorSubcoreMesh(core_axis_name="c", subcore_axis_name="s")


@functools.partial(
    pl.kernel,
    out_type=jax.ShapeDtypeStruct((NP, F), jnp.float32),
    mesh=_sc_mesh,
    scratch_types=[
        pltpu.VMEM((CN * DEG,), jnp.int32),
        pltpu.VMEM((CN * DEG, F), jnp.float32),
        pltpu.VMEM((CN, F), jnp.float32),
        pltpu.SemaphoreType.DMA,
    ],
)
def _gather_max(y_hbm, idx_hbm, out_hbm, idx_v, rows_v, out_v, sem):
    wid = lax.axis_index("s") * NC + lax.axis_index("c")
    node_base = wid * PW

    def chunk_body(ci, _):
        nb = node_base + ci * CN
        pltpu.sync_copy(idx_hbm.at[pl.ds(nb * DEG, CN * DEG)], idx_v)
        pltpu.async_copy(y_hbm.at[idx_v], rows_v, sem).wait()

        def node_body(j, _):
            r0 = j * DEG
            for c in range(F // L):
                acc = jnp.zeros((L,), jnp.float32)
                for d in range(DEG):
                    acc = jnp.maximum(acc, rows_v[r0 + d, pl.ds(c * L, L)])
                out_v[j, pl.ds(c * L, L)] = acc
            return 0

        lax.fori_loop(0, CN, node_body, 0, unroll=False)
        pltpu.sync_copy(out_v, out_hbm.at[pl.ds(nb, CN)])
        return 0

    lax.fori_loop(0, NCH, chunk_body, 0, unroll=False)


def kernel(x, neigh, W, b):
    y = _matmul(x, W, b)
    idx = neigh.astype(jnp.int32)
    idx = jnp.pad(idx, ((0, NP - N), (0, 0))).reshape(NP * DEG)
    out = _gather_max(y, idx)
    return out[:N]


# speedup vs baseline: 1.5157x; 1.0593x over previous
import functools

import jax
import jax.numpy as jnp
from jax import lax
from jax.experimental import pallas as pl
from jax.experimental.pallas import tpu as pltpu
from jax.experimental.pallas import tpu_sc as plsc

N = 10000
DEG = 32
F = 128
LF = 16

NC, NS = 2, 16
NW = NC * NS
NP = 10240
PW = NP // NW
CN = 8
NCH = PW // CN

BM = 1000


def _mm_body(x_ref, w_ref, b_ref, o_ref):
    o_ref[...] = (
        jnp.dot(x_ref[...], w_ref[...], preferred_element_type=jnp.float32)
        + b_ref[...]
    )


def _matmul(x, W, b):
    return pl.pallas_call(
        _mm_body,
        grid=(N // BM,),
        in_specs=[
            pl.BlockSpec((BM, F), lambda i: (i, 0)),
            pl.BlockSpec((F, F), lambda i: (0, 0)),
            pl.BlockSpec((1, F), lambda i: (0, 0)),
        ],
        out_specs=pl.BlockSpec((BM, F), lambda i: (i, 0)),
        out_shape=jax.ShapeDtypeStruct((N, F), jnp.float32),
    )(x, W, b.reshape(1, F))


_sc_mesh = plsc.VectorSubcoreMesh(core_axis_name="c", subcore_axis_name="s")


@functools.partial(
    pl.kernel,
    out_type=jax.ShapeDtypeStruct((NP, F), jnp.float32),
    mesh=_sc_mesh,
    scratch_types=[
        pltpu.VMEM((PW * DEG,), jnp.int32),
        pltpu.VMEM((CN * DEG, F), jnp.float32),
        pltpu.VMEM((CN * DEG, F), jnp.float32),
        pltpu.VMEM((CN, F), jnp.float32),
        pltpu.SemaphoreType.DMA,
        pltpu.SemaphoreType.DMA,
    ],
)
def _gather_max(y_hbm, idx_hbm, out_hbm, idx_all, rows0, rows1, outb, s0, s1):
    wid = lax.axis_index("s") * NC + lax.axis_index("c")
    base = wid * PW
    pltpu.sync_copy(idx_hbm.at[pl.ds(base * DEG, PW * DEG)], idx_all)

    def idxs(ci):
        return idx_all.at[pl.ds(ci * CN * DEG, CN * DEG)]

    def compute(rows_v, ci):
        def node_body(j, _):
            r0 = j * DEG
            for c in range(F // LF):
                acc = jnp.zeros((LF,), jnp.float32)
                for d in range(DEG):
                    acc = jnp.maximum(acc, rows_v[r0 + d, pl.ds(c * LF, LF)])
                outb[j, pl.ds(c * LF, LF)] = acc
            return 0

        lax.fori_loop(0, CN, node_body, 0, unroll=False)
        pltpu.sync_copy(outb, out_hbm.at[pl.ds(base + ci * CN, CN)])

    pltpu.async_copy(y_hbm.at[idxs(0)], rows0, s0)

    def pair_body(i, _):
        ci0 = i * 2
        pltpu.async_copy(y_hbm.at[idxs(ci0 + 1)], rows1, s1)
        pltpu.make_async_copy(y_hbm.at[idxs(ci0)], rows0, s0).wait()
        compute(rows0, ci0)

        @pl.when(ci0 + 2 < NCH)
        def _():
            pltpu.async_copy(y_hbm.at[idxs(ci0 + 2)], rows0, s0)

        pltpu.make_async_copy(y_hbm.at[idxs(ci0 + 1)], rows1, s1).wait()
        compute(rows1, ci0 + 1)
        return 0

    lax.fori_loop(0, NCH // 2, pair_body, 0, unroll=False)


def kernel(x, neigh, W, b):
    y = _matmul(x, W, b)
    idx = neigh.astype(jnp.int32)
    idx = jnp.pad(idx, ((0, NP - N), (0, 0))).reshape(NP * DEG)
    out = _gather_max(y, idx)
    return out[:N]
